# Initial kernel scaffold; baseline (speedup 1.0000x reference)
import jax
import jax.numpy as jnp
from jax.experimental import pallas as pl


def kernel(char_logits, radical_logits, structure, stroke_count, stroke_types, stroke_type_sig, W1, b1, W2, b2, reranker_weight, radical_mask, structure_label, stroke_count_label):
    raise NotImplementedError("write your pallas kernel here")


# speedup vs baseline: 1.0346x; 1.0346x over previous
import jax
import jax.numpy as jnp
from jax.experimental import pallas as pl

_B = 1024
_C = 100000
_R = 214
_NS = 13
_NSC = 30
_NST = 6
_K = 256
_H = 256
_BB = 8


def _rerank_block(top_logits_ref, radical_logits_ref, structure_ref,
                  stroke_count_ref, stroke_types_ref, cand_mask_ref,
                  cand_sig_ref, cand_structure_ref, cand_stroke_ref,
                  w1_ref, b1_ref, w2_ref, b2_ref, rw_ref, out_ref):
    rl = radical_logits_ref[...]
    radical_preds = (rl > 0.0).astype(jnp.float32)
    mask = cand_mask_ref[...].astype(jnp.float32)
    rp = radical_preds[:, None, :]
    detected = jnp.sum(rp * mask, axis=-1)
    counts = jnp.maximum(jnp.sum(mask, axis=-1), 1.0)
    match_ratio = detected / counts
    num_detected = jnp.sum(radical_preds, axis=-1, keepdims=True)
    false_alarms = jnp.sum(rp * (1.0 - mask), axis=-1)
    false_ratio = false_alarms / jnp.maximum(num_detected, 1.0)

    probs = jax.nn.softmax(structure_ref[...], axis=-1)
    cs = cand_structure_ref[...]
    ns_iota = jax.lax.broadcasted_iota(jnp.int32, (1, 1, _NS), 2)
    eq = (cs[:, :, None] == ns_iota).astype(jnp.float32)
    structure_match = jnp.sum(eq * probs[:, None, :], axis=-1)

    sc = stroke_count_ref[...]
    mx = jnp.max(sc, axis=-1, keepdims=True)
    idx_iota = jax.lax.broadcasted_iota(jnp.int32, sc.shape, 1)
    stroke_pred = jnp.min(jnp.where(sc == mx, idx_iota, _NSC), axis=-1)
    cstk = cand_stroke_ref[...]
    stroke_distance = jnp.abs(stroke_pred[:, None] - cstk).astype(jnp.float32) / 29.0

    st = stroke_types_ref[...]
    pn = st / jnp.maximum(jnp.sqrt(jnp.sum(st * st, -1, keepdims=True)), 1e-12)
    sig = cand_sig_ref[...]
    sgn = sig / jnp.maximum(jnp.sqrt(jnp.sum(sig * sig, -1, keepdims=True)), 1e-12)
    cos = jnp.sum(pn[:, None, :] * sgn, axis=-1)

    feats = (match_ratio, false_ratio, structure_match, stroke_distance, cos)
    acc = jnp.broadcast_to(b1_ref[...].reshape(1, 1, _H), (_BB, _K, _H))
    for f in range(5):
        wrow = w1_ref[f:f + 1, :].reshape(1, 1, _H)
        acc = acc + feats[f][:, :, None] * wrow
    h = jnp.maximum(acc, 0.0)
    score = jnp.sum(h * w2_ref[...].reshape(1, 1, _H), axis=-1) + b2_ref[0, 0]

    out_ref[...] = top_logits_ref[...] + rw_ref[0, 0] * score


def kernel(char_logits, radical_logits, structure, stroke_count, stroke_types,
           stroke_type_sig, W1, b1, W2, b2, reranker_weight,
           radical_mask, structure_label, stroke_count_label):
    top_logits, top_indices = jax.lax.top_k(char_logits, _K)
    cand_mask = jnp.take(radical_mask, top_indices, axis=0)
    cand_structure = jnp.take(structure_label, top_indices, axis=0).astype(jnp.int32)
    cand_stroke = jnp.take(stroke_count_label, top_indices, axis=0).astype(jnp.int32)
    cand_sig = jnp.take(stroke_type_sig, top_indices, axis=0)

    grid = (_B // _BB,)
    combined_top = pl.pallas_call(
        _rerank_block,
        grid=grid,
        in_specs=[
            pl.BlockSpec((_BB, _K), lambda i: (i, 0)),
            pl.BlockSpec((_BB, _R), lambda i: (i, 0)),
            pl.BlockSpec((_BB, _NS), lambda i: (i, 0)),
            pl.BlockSpec((_BB, _NSC), lambda i: (i, 0)),
            pl.BlockSpec((_BB, _NST), lambda i: (i, 0)),
            pl.BlockSpec((_BB, _K, _R), lambda i: (i, 0, 0)),
            pl.BlockSpec((_BB, _K, _NST), lambda i: (i, 0, 0)),
            pl.BlockSpec((_BB, _K), lambda i: (i, 0)),
            pl.BlockSpec((_BB, _K), lambda i: (i, 0)),
            pl.BlockSpec((5, _H), lambda i: (0, 0)),
            pl.BlockSpec((1, _H), lambda i: (0, 0)),
            pl.BlockSpec((1, _H), lambda i: (0, 0)),
            pl.BlockSpec((1, 1), lambda i: (0, 0)),
            pl.BlockSpec((1, 1), lambda i: (0, 0)),
        ],
        out_specs=pl.BlockSpec((_BB, _K), lambda i: (i, 0)),
        out_shape=jax.ShapeDtypeStruct((_B, _K), jnp.float32),
    )(top_logits, radical_logits, structure, stroke_count, stroke_types,
      cand_mask, cand_sig, cand_structure, cand_stroke,
      W1, b1.reshape(1, _H), W2.reshape(1, _H), b2.reshape(1, 1),
      jnp.reshape(reranker_weight, (1, 1)).astype(jnp.float32))

    combined_logits = char_logits.at[jnp.arange(_B)[:, None], top_indices].set(combined_top)
    return combined_logits
